# baseline (speedup 1.0000x reference)
import functools

import jax
import jax.numpy as jnp
from jax import lax
from jax.experimental import pallas as pl
from jax.experimental.pallas import tpu as pltpu
from jax.experimental.pallas import tpu_sc as plsc

N = 2048
E = 65536
C_AE = 256
C_PE = 64
C_SE = 64
K_TOP = 32

BR = 256
NEG = -3.0e38


_EPT = E // 16
_CHUNK = (N * N) // 4
_SLICE = _CHUNK // 16
_NDUM = 64
_ZB = 32768


def _sc_cnt_kernel(edge_hbm, sd_hbm, src_v, dst_v, idxb, valb, zb, chunk_sh, sem):
    cid = lax.axis_index("c")
    sid = lax.axis_index("s")

    ebase = sid * _EPT
    pltpu.sync_copy(edge_hbm.at[0, pl.ds(ebase, _EPT)], src_v)
    pltpu.sync_copy(edge_hbm.at[1, pl.ds(ebase, _EPT)], dst_v)

    def _zero_zb(i, _):
        zb[pl.ds(i * 16, 16)] = jnp.zeros((16,), jnp.float32)
        return 0

    lax.fori_loop(0, _ZB // 16, _zero_zb, 0)

    lanes = lax.iota(jnp.int32, 16)

    for j in range(2):
        base_dyn = (cid * 2 + j) * _CHUNK
        for t in range(_SLICE // _ZB):
            pltpu.sync_copy(zb, chunk_sh.at[pl.ds(sid * _SLICE + t * _ZB, _ZB)])

        @pl.when(sid == 0)
        def _():
            pltpu.sync_copy(zb.at[pl.ds(0, _NDUM)], chunk_sh.at[pl.ds(_CHUNK, _NDUM)])

        plsc.subcore_barrier()

        copies = []
        for g in range(32):
            def _fill(i, _, g=g):
                s16 = src_v[pl.ds(g * 128 + i * 16, 16)]
                d16 = dst_v[pl.ds(g * 128 + i * 16, 16)]
                flat = lax.shift_left(s16, 11) + d16
                rel = flat - base_dyn
                inm = (rel >= 0) & (rel < _CHUNK)
                idx16 = jnp.where(inm, rel, _CHUNK + lanes * 4)
                val16 = jnp.where(inm, jnp.float32(1.0), jnp.float32(0.0))
                idxb[g, pl.ds(i * 16, 16)] = idx16
                valb[g, pl.ds(i * 16, 16)] = val16
                return 0

            lax.fori_loop(0, 8, _fill, 0)
            copies.append(
                pltpu.async_copy(valb.at[g], chunk_sh.at[idxb.at[g]], sem, add=True)
            )
        for cp in copies:
            cp.wait()

        plsc.subcore_barrier()

        pltpu.sync_copy(chunk_sh.at[pl.ds(sid * _SLICE, _SLICE)],
                        sd_hbm.at[pl.ds(base_dyn + sid * _SLICE, _SLICE)])

        plsc.subcore_barrier()


def _sc_cnt(edge_index):
    mesh = plsc.VectorSubcoreMesh(core_axis_name="c", subcore_axis_name="s")
    f = functools.partial(
        pl.kernel,
        mesh=mesh,
        out_type=jax.ShapeDtypeStruct((N * N,), jnp.float32),
        scratch_types=[
            pltpu.VMEM((_EPT,), jnp.int32),
            pltpu.VMEM((_EPT,), jnp.int32),
            pltpu.VMEM((32, 128), jnp.int32),
            pltpu.VMEM((32, 128), jnp.float32),
            pltpu.VMEM((_ZB,), jnp.float32),
            pltpu.VMEM_SHARED((_CHUNK + _NDUM,), jnp.float32),
            pltpu.SemaphoreType.DMA,
        ],
    )(_sc_cnt_kernel)
    return f(edge_index).reshape(N, N)


def _dot(a, b):
    return lax.dot_general(a, b, (((1,), (0,)), ((), ())),
                           preferred_element_type=jnp.float32)


def _dot_hi(a, b):
    return lax.dot_general(a, b, (((1,), (0,)), ((), ())),
                           precision=lax.Precision.HIGHEST,
                           preferred_element_type=jnp.float32)


def _prep_kernel(ae_ref, pe_ref, se_ref, *rest):
    wrefs = rest[:27]
    outs = rest[27:]
    xs = (ae_ref, pe_ref, se_ref)
    for c in range(3):
        W1, b1, W2, b2, Wq, bq, Wk, bk, vsd = wrefs[c * 9:(c + 1) * 9]
        x = xs[c][...]
        h1 = jnp.maximum(_dot(x, W1[...]) + b1[...], 0.0)
        xh = _dot(h1, W2[...]) + b2[...]
        q = _dot(xh, Wq[...]) + bq[...]
        k = _dot(xh, Wk[...]) + bk[...]
        sc = _dot(xh, vsd[...])
        o_h, o_q, o_k, o_sc = outs[c * 4:(c + 1) * 4]
        o_h[...] = xh
        o_q[...] = q
        o_k[...] = k
        o_sc[...] = sc


def _prep(ae, pe, se, p):
    chans = ("ae", "pe", "se")
    dims = {"ae": C_AE, "pe": C_PE, "se": C_SE}
    weights = []
    w_specs = []
    outs = []
    o_specs = []
    for name in chans:
        c = dims[name]
        vsd = p[f"{name}_gat_W"] @ jnp.stack(
            [p[f"{name}_gat_asrc"], p[f"{name}_gat_adst"]], axis=1)
        ws = [
            p[f"{name}_enc_W1"], p[f"{name}_enc_b1"].reshape(1, 2 * c),
            p[f"{name}_enc_W2"], p[f"{name}_enc_b2"].reshape(1, c),
            p[f"{name}_mha_Wq"], p[f"{name}_mha_bq"].reshape(1, c),
            p[f"{name}_mha_Wk"], p[f"{name}_mha_bk"].reshape(1, c),
            vsd,
        ]
        weights += ws
        w_specs += [pl.BlockSpec(w.shape, lambda i: (0, 0)) for w in ws]
        outs += [
            jax.ShapeDtypeStruct((N, c), jnp.float32),
            jax.ShapeDtypeStruct((N, c), jnp.float32),
            jax.ShapeDtypeStruct((N, c), jnp.float32),
            jax.ShapeDtypeStruct((N, 2), jnp.float32),
        ]
        o_specs += [
            pl.BlockSpec((BR, c), lambda i: (i, 0)),
            pl.BlockSpec((BR, c), lambda i: (i, 0)),
            pl.BlockSpec((BR, c), lambda i: (i, 0)),
            pl.BlockSpec((BR, 2), lambda i: (i, 0)),
        ]
    x_specs = [
        pl.BlockSpec((BR, C_AE), lambda i: (i, 0)),
        pl.BlockSpec((BR, C_PE), lambda i: (i, 0)),
        pl.BlockSpec((BR, C_SE), lambda i: (i, 0)),
    ]
    res = pl.pallas_call(
        _prep_kernel,
        grid=(N // BR,),
        in_specs=x_specs + w_specs,
        out_specs=o_specs,
        out_shape=outs,
    )(ae, pe, se, *weights)
    out = {}
    for ci, name in enumerate(chans):
        out[name] = tuple(res[ci * 4:(ci + 1) * 4])
    return out


def _dot_t(a, b):
    return lax.dot_general(
        a, b, (((0,), (0,)), ((), ())),
        precision=lax.Precision.HIGHEST,
        preferred_element_type=jnp.float32)


def _local_kernel(cnt_ref, ssrc_pe, ssrc_se, ssrc_ae, sdrow_pe, sdrow_se,
                  sdrow_ae, aeh_ref, wl_ref, ones_ref, coef_ref, out_ref):
    cntb = cnt_ref[...]
    pos = cntb > 0.0
    aw = None
    coefs = (coef_ref[2], coef_ref[0], coef_ref[1])
    for ci, (ssrc, sdrow) in enumerate(((ssrc_ae, sdrow_ae), (ssrc_pe, sdrow_pe),
                                        (ssrc_se, sdrow_se))):
        e = ssrc[...] + sdrow[...]
        e = jnp.where(e > 0.0, e, 0.2 * e)
        m = jnp.max(jnp.where(pos, e, NEG), axis=0, keepdims=True)
        ex = jnp.where(pos, jnp.exp(e - m), 0.0)
        s = jnp.sum(cntb * ex, axis=0, keepdims=True)
        alpha = ex / (s + 1e-16)
        aw = coefs[ci] * alpha if aw is None else aw + coefs[ci] * alpha
    w = cntb * aw
    num = _dot_t(w, aeh_ref[...])
    deg = _dot_t(cntb, ones_ref[...])
    out_ref[...] = _dot(num / jnp.maximum(deg, 1.0), wl_ref[...])


def _local(cnt_sd, prep, w_local, lcoef):
    ssrcs = []
    sdrows = []
    for name in ("pe", "se", "ae"):
        sc = prep[name][3]
        ssrcs.append(sc[:, 0:1])
        sdrows.append(sc[:, 1].reshape(1, N))
    ones = jnp.ones((N, 1), jnp.float32)
    return pl.pallas_call(
        _local_kernel,
        grid=(N // BR,),
        in_specs=[
            pl.BlockSpec((N, BR), lambda j: (0, j)),
            pl.BlockSpec((N, 1), lambda j: (0, 0)),
            pl.BlockSpec((N, 1), lambda j: (0, 0)),
            pl.BlockSpec((N, 1), lambda j: (0, 0)),
            pl.BlockSpec((1, BR), lambda j: (0, j)),
            pl.BlockSpec((1, BR), lambda j: (0, j)),
            pl.BlockSpec((1, BR), lambda j: (0, j)),
            pl.BlockSpec((N, C_AE), lambda j: (0, 0)),
            pl.BlockSpec((C_AE, C_AE), lambda j: (0, 0)),
            pl.BlockSpec((N, 1), lambda j: (0, 0)),
            pl.BlockSpec(memory_space=pltpu.SMEM),
        ],
        out_specs=pl.BlockSpec((BR, C_AE), lambda j: (j, 0)),
        out_shape=jax.ShapeDtypeStruct((N, C_AE), jnp.float32),
    )(cnt_sd, ssrcs[0], ssrcs[1], ssrcs[2], sdrows[0], sdrows[1], sdrows[2],
      prep["ae"][0], w_local, ones, lcoef)


def _row_softmax(q, k, scale):
    s = lax.dot_general(q, k, (((1,), (1,)), ((), ())),
                        preferred_element_type=jnp.float32) * scale
    m = jnp.max(s, axis=1, keepdims=True)
    p = jnp.exp(s - m)
    return p / jnp.sum(p, axis=1, keepdims=True)


def _attn_kernel(qpe, kpe, qse, kse, qae, kae, cnt_ref, ones_ref, coef_ref,
                 g_ref, col_ref):
    br = qpe.shape[0]
    a_g = coef_ref[0]
    b_g = coef_ref[1]
    c_g = coef_ref[2]
    ones = ones_ref[...]
    pe_g = _row_softmax(qpe[...], kpe[...], 1.0 / 8.0)
    se_g = _row_softmax(qse[...], kse[...], 1.0 / 8.0)
    ae_g = _row_softmax(qae[...], kae[...], 1.0 / 16.0)
    sample = a_g * pe_g + b_g * se_g
    sample = jnp.where(cnt_ref[...] > 0.0, 0.0, sample)

    del ones

    def _count(m):
        return jnp.sum(m.astype(jnp.float32), axis=1, keepdims=True)

    v = lax.bitcast_convert_type(sample, jnp.int32)
    th = jnp.zeros((br, 1), jnp.int32)
    for b in range(29, -1, -1):
        cand = th | (1 << b)
        cge = _count(v >= cand)
        th = jnp.where(cge >= float(K_TOP), cand, th)
    gt = v > th
    eq = v == th
    ngt = _count(gt)
    need = float(K_TOP) - ngt
    cols = lax.broadcasted_iota(jnp.int32, (br, N), 1)
    jt = jnp.zeros((br, 1), jnp.int32)
    for b in range(10, -1, -1):
        cand = jt | (1 << b)
        c = _count(eq & (cols <= cand))
        jt = jnp.where(c <= need, cand, jt)
    mask = gt | (eq & (cols <= jt) & (need > 0.0))
    maskf = mask.astype(jnp.float32)
    g = (0.5 * a_g + 0.5 * b_g) * (sample * maskf) + c_g * (ae_g * maskf)
    g_ref[...] = g.astype(jnp.bfloat16)
    col_ref[...] = jnp.sum(g, axis=1, keepdims=True)


def _attn(cnt_sd, prep, gcoef):
    qpe, kpe = prep["pe"][1], prep["pe"][2]
    qse, kse = prep["se"][1], prep["se"][2]
    qae, kae = prep["ae"][1], prep["ae"][2]
    ones = jnp.ones((N, 1), jnp.float32)
    BA = 512
    return pl.pallas_call(
        _attn_kernel,
        grid=(N // BA,),
        in_specs=[
            pl.BlockSpec((BA, C_PE), lambda i: (i, 0)),
            pl.BlockSpec((N, C_PE), lambda i: (0, 0)),
            pl.BlockSpec((BA, C_SE), lambda i: (i, 0)),
            pl.BlockSpec((N, C_SE), lambda i: (0, 0)),
            pl.BlockSpec((BA, C_AE), lambda i: (i, 0)),
            pl.BlockSpec((N, C_AE), lambda i: (0, 0)),
            pl.BlockSpec((BA, N), lambda i: (i, 0)),
            pl.BlockSpec((N, 1), lambda i: (0, 0)),
            pl.BlockSpec(memory_space=pltpu.SMEM),
        ],
        out_specs=[
            pl.BlockSpec((BA, N), lambda i: (i, 0)),
            pl.BlockSpec((BA, 1), lambda i: (i, 0)),
        ],
        out_shape=[
            jax.ShapeDtypeStruct((N, N), jnp.bfloat16),
            jax.ShapeDtypeStruct((N, 1), jnp.float32),
        ],
    )(qpe, kpe, qse, kse, qae, kae, cnt_sd, ones, gcoef)


def _final_kernel(g_ref, colrow_ref, aeh_ref, wg_ref, outl_ref, out_ref):
    gn = g_ref[...].astype(jnp.float32) / colrow_ref[...]
    out_ref[...] = outl_ref[...] + _dot(_dot(gn, aeh_ref[...]), wg_ref[...])


def _final(g, col, aeh, w_global, out_l):
    col_row = col.reshape(1, N)
    return pl.pallas_call(
        _final_kernel,
        grid=(N // BR,),
        in_specs=[
            pl.BlockSpec((BR, N), lambda i: (i, 0)),
            pl.BlockSpec((1, N), lambda i: (0, 0)),
            pl.BlockSpec((N, C_AE), lambda i: (0, 0)),
            pl.BlockSpec((C_AE, C_AE), lambda i: (0, 0)),
            pl.BlockSpec((BR, C_AE), lambda i: (i, 0)),
        ],
        out_specs=pl.BlockSpec((BR, C_AE), lambda i: (i, 0)),
        out_shape=jax.ShapeDtypeStruct((N, C_AE), jnp.float32),
    )(g, col_row, aeh, w_global, out_l)


def _tc_forward(ae, pe, se, cnt_sd, p):
    prep = _prep(ae, pe, se, p)
    lcoef = jnp.stack([p["a_l"], p["b_l"], p["c_l"]])
    gcoef = jnp.stack([p["a_g"], p["b_g"], p["c_g"]])
    out_l = _local(cnt_sd, prep, p["W_local"], lcoef)
    g, col = _attn(cnt_sd, prep, gcoef)
    return _final(g, col, prep["ae"][0], p["W_global"], out_l)


def kernel(ae, pe, se, edge_index, K, params):
    del K
    cnt_sd = _sc_cnt(edge_index)
    return _tc_forward(ae, pe, se, cnt_sd, params)


# speedup vs baseline: 1.0821x; 1.0821x over previous
import functools

import jax
import jax.numpy as jnp
from jax import lax
from jax.experimental import pallas as pl
from jax.experimental.pallas import tpu as pltpu
from jax.experimental.pallas import tpu_sc as plsc

N = 2048
E = 65536
C_AE = 256
C_PE = 64
C_SE = 64
K_TOP = 32

BR = 256
NEG = -3.0e38


_EPT = E // 16
_CHUNK = (N * N) // 4
_SLICE = _CHUNK // 16
_NDUM = 64
_ZB = 32768


def _sc_cnt_kernel(edge_hbm, sd_hbm, src_v, dst_v, idxb, valb, zb, chunk_sh, sem):
    cid = lax.axis_index("c")
    sid = lax.axis_index("s")

    ebase = sid * _EPT
    pltpu.sync_copy(edge_hbm.at[0, pl.ds(ebase, _EPT)], src_v)
    pltpu.sync_copy(edge_hbm.at[1, pl.ds(ebase, _EPT)], dst_v)

    def _zero_zb(i, _):
        zb[pl.ds(i * 16, 16)] = jnp.zeros((16,), jnp.float32)
        return 0

    lax.fori_loop(0, _ZB // 16, _zero_zb, 0)

    lanes = lax.iota(jnp.int32, 16)

    for j in range(2):
        base_dyn = (cid * 2 + j) * _CHUNK
        for t in range(_SLICE // _ZB):
            pltpu.sync_copy(zb, chunk_sh.at[pl.ds(sid * _SLICE + t * _ZB, _ZB)])

        @pl.when(sid == 0)
        def _():
            pltpu.sync_copy(zb.at[pl.ds(0, _NDUM)], chunk_sh.at[pl.ds(_CHUNK, _NDUM)])

        plsc.subcore_barrier()

        copies = []
        for g in range(32):
            def _fill(i, _, g=g):
                s16 = src_v[pl.ds(g * 128 + i * 16, 16)]
                d16 = dst_v[pl.ds(g * 128 + i * 16, 16)]
                flat = lax.shift_left(s16, 11) + d16
                rel = flat - base_dyn
                inm = (rel >= 0) & (rel < _CHUNK)
                idx16 = jnp.where(inm, rel, _CHUNK + lanes * 4)
                val16 = jnp.where(inm, jnp.float32(1.0), jnp.float32(0.0))
                idxb[g, pl.ds(i * 16, 16)] = idx16
                valb[g, pl.ds(i * 16, 16)] = val16
                return 0

            lax.fori_loop(0, 8, _fill, 0)
            copies.append(
                pltpu.async_copy(valb.at[g], chunk_sh.at[idxb.at[g]], sem, add=True)
            )
        for cp in copies:
            cp.wait()

        plsc.subcore_barrier()

        pltpu.sync_copy(chunk_sh.at[pl.ds(sid * _SLICE, _SLICE)],
                        sd_hbm.at[pl.ds(base_dyn + sid * _SLICE, _SLICE)])

        plsc.subcore_barrier()


def _sc_cnt(edge_index):
    mesh = plsc.VectorSubcoreMesh(core_axis_name="c", subcore_axis_name="s")
    f = functools.partial(
        pl.kernel,
        mesh=mesh,
        out_type=jax.ShapeDtypeStruct((N * N,), jnp.float32),
        scratch_types=[
            pltpu.VMEM((_EPT,), jnp.int32),
            pltpu.VMEM((_EPT,), jnp.int32),
            pltpu.VMEM((32, 128), jnp.int32),
            pltpu.VMEM((32, 128), jnp.float32),
            pltpu.VMEM((_ZB,), jnp.float32),
            pltpu.VMEM_SHARED((_CHUNK + _NDUM,), jnp.float32),
            pltpu.SemaphoreType.DMA,
        ],
    )(_sc_cnt_kernel)
    return f(edge_index).reshape(N, N)


def _dot(a, b):
    return lax.dot_general(a, b, (((1,), (0,)), ((), ())),
                           preferred_element_type=jnp.float32)


def _dot_hi(a, b):
    return lax.dot_general(a, b, (((1,), (0,)), ((), ())),
                           precision=lax.Precision.HIGHEST,
                           preferred_element_type=jnp.float32)


def _prep_kernel(ae_ref, pe_ref, se_ref, *rest):
    wrefs = rest[:27]
    outs = rest[27:]
    xs = (ae_ref, pe_ref, se_ref)
    for c in range(3):
        W1, b1, W2, b2, Wq, bq, Wk, bk, vsd = wrefs[c * 9:(c + 1) * 9]
        x = xs[c][...]
        h1 = jnp.maximum(_dot(x, W1[...]) + b1[...], 0.0)
        xh = _dot(h1, W2[...]) + b2[...]
        q = _dot(xh, Wq[...]) + bq[...]
        k = _dot(xh, Wk[...]) + bk[...]
        sc = _dot(xh, vsd[...])
        o_h, o_q, o_k, o_sc = outs[c * 4:(c + 1) * 4]
        o_h[...] = xh
        o_q[...] = q
        o_k[...] = k
        o_sc[...] = sc


def _prep(ae, pe, se, p):
    chans = ("ae", "pe", "se")
    dims = {"ae": C_AE, "pe": C_PE, "se": C_SE}
    weights = []
    w_specs = []
    outs = []
    o_specs = []
    for name in chans:
        c = dims[name]
        vsd = p[f"{name}_gat_W"] @ jnp.stack(
            [p[f"{name}_gat_asrc"], p[f"{name}_gat_adst"]], axis=1)
        ws = [
            p[f"{name}_enc_W1"], p[f"{name}_enc_b1"].reshape(1, 2 * c),
            p[f"{name}_enc_W2"], p[f"{name}_enc_b2"].reshape(1, c),
            p[f"{name}_mha_Wq"], p[f"{name}_mha_bq"].reshape(1, c),
            p[f"{name}_mha_Wk"], p[f"{name}_mha_bk"].reshape(1, c),
            vsd,
        ]
        weights += ws
        w_specs += [pl.BlockSpec(w.shape, lambda i: (0, 0)) for w in ws]
        outs += [
            jax.ShapeDtypeStruct((N, c), jnp.float32),
            jax.ShapeDtypeStruct((N, c), jnp.float32),
            jax.ShapeDtypeStruct((N, c), jnp.float32),
            jax.ShapeDtypeStruct((N, 2), jnp.float32),
        ]
        o_specs += [
            pl.BlockSpec((BR, c), lambda i: (i, 0)),
            pl.BlockSpec((BR, c), lambda i: (i, 0)),
            pl.BlockSpec((BR, c), lambda i: (i, 0)),
            pl.BlockSpec((BR, 2), lambda i: (i, 0)),
        ]
    x_specs = [
        pl.BlockSpec((BR, C_AE), lambda i: (i, 0)),
        pl.BlockSpec((BR, C_PE), lambda i: (i, 0)),
        pl.BlockSpec((BR, C_SE), lambda i: (i, 0)),
    ]
    res = pl.pallas_call(
        _prep_kernel,
        grid=(N // BR,),
        in_specs=x_specs + w_specs,
        out_specs=o_specs,
        out_shape=outs,
    )(ae, pe, se, *weights)
    out = {}
    for ci, name in enumerate(chans):
        out[name] = tuple(res[ci * 4:(ci + 1) * 4])
    return out


def _dot_t(a, b):
    return lax.dot_general(
        a, b, (((0,), (0,)), ((), ())),
        precision=lax.Precision.HIGHEST,
        preferred_element_type=jnp.float32)


def _local_kernel(cnt_ref, ssrc_pe, ssrc_se, ssrc_ae, sdrow_pe, sdrow_se,
                  sdrow_ae, aeh_ref, wl_ref, ones_ref, coef_ref, out_ref):
    cntb = cnt_ref[...]
    pos = cntb > 0.0
    aw = None
    coefs = (coef_ref[2], coef_ref[0], coef_ref[1])
    for ci, (ssrc, sdrow) in enumerate(((ssrc_ae, sdrow_ae), (ssrc_pe, sdrow_pe),
                                        (ssrc_se, sdrow_se))):
        e = ssrc[...] + sdrow[...]
        e = jnp.where(e > 0.0, e, 0.2 * e)
        m = jnp.max(jnp.where(pos, e, NEG), axis=0, keepdims=True)
        ex = jnp.where(pos, jnp.exp(e - m), 0.0)
        s = jnp.sum(cntb * ex, axis=0, keepdims=True)
        alpha = ex / (s + 1e-16)
        aw = coefs[ci] * alpha if aw is None else aw + coefs[ci] * alpha
    w = cntb * aw
    num = _dot_t(w, aeh_ref[...])
    deg = _dot_t(cntb, ones_ref[...])
    out_ref[...] = _dot(num / jnp.maximum(deg, 1.0), wl_ref[...])


def _local(cnt_sd, prep, w_local, lcoef):
    ssrcs = []
    sdrows = []
    for name in ("pe", "se", "ae"):
        sc = prep[name][3]
        ssrcs.append(sc[:, 0:1])
        sdrows.append(sc[:, 1].reshape(1, N))
    ones = jnp.ones((N, 1), jnp.float32)
    return pl.pallas_call(
        _local_kernel,
        grid=(N // BR,),
        in_specs=[
            pl.BlockSpec((N, BR), lambda j: (0, j)),
            pl.BlockSpec((N, 1), lambda j: (0, 0)),
            pl.BlockSpec((N, 1), lambda j: (0, 0)),
            pl.BlockSpec((N, 1), lambda j: (0, 0)),
            pl.BlockSpec((1, BR), lambda j: (0, j)),
            pl.BlockSpec((1, BR), lambda j: (0, j)),
            pl.BlockSpec((1, BR), lambda j: (0, j)),
            pl.BlockSpec((N, C_AE), lambda j: (0, 0)),
            pl.BlockSpec((C_AE, C_AE), lambda j: (0, 0)),
            pl.BlockSpec((N, 1), lambda j: (0, 0)),
            pl.BlockSpec(memory_space=pltpu.SMEM),
        ],
        out_specs=pl.BlockSpec((BR, C_AE), lambda j: (j, 0)),
        out_shape=jax.ShapeDtypeStruct((N, C_AE), jnp.float32),
    )(cnt_sd, ssrcs[0], ssrcs[1], ssrcs[2], sdrows[0], sdrows[1], sdrows[2],
      prep["ae"][0], w_local, ones, lcoef)


def _row_softmax(q, k, scale):
    s = lax.dot_general(q, k, (((1,), (1,)), ((), ())),
                        preferred_element_type=jnp.float32) * scale
    m = jnp.max(s, axis=1, keepdims=True)
    p = jnp.exp(s - m)
    return p / jnp.sum(p, axis=1, keepdims=True)


def _attn_kernel(qpe, kpe, qse, kse, qae, kae, cnt_ref, ones_ref, coef_ref,
                 g_ref, col_ref):
    br = qpe.shape[0]
    a_g = coef_ref[0]
    b_g = coef_ref[1]
    c_g = coef_ref[2]
    ones = ones_ref[...]
    pe_g = _row_softmax(qpe[...], kpe[...], 1.0 / 8.0)
    se_g = _row_softmax(qse[...], kse[...], 1.0 / 8.0)
    ae_g = _row_softmax(qae[...], kae[...], 1.0 / 16.0)
    sample = a_g * pe_g + b_g * se_g
    sample = jnp.where(cnt_ref[...] > 0.0, 0.0, sample)

    del ones

    def _count(m):
        return jnp.sum(m.astype(jnp.float32), axis=1, keepdims=True)

    v = lax.bitcast_convert_type(sample, jnp.int32)
    th = jnp.zeros((br, 1), jnp.int32)
    for b in range(29, -1, -1):
        cand = th | (1 << b)
        cge = _count(v >= cand)
        th = jnp.where(cge >= float(K_TOP), cand, th)
    gt = v > th
    eq = v == th
    ngt = _count(gt)
    need = float(K_TOP) - ngt
    cols = lax.broadcasted_iota(jnp.int32, (br, N), 1)
    jt = jnp.zeros((br, 1), jnp.int32)
    for b in range(10, -1, -1):
        cand = jt | (1 << b)
        c = _count(eq & (cols <= cand))
        jt = jnp.where(c <= need, cand, jt)
    mask = gt | (eq & (cols <= jt) & (need > 0.0))
    maskf = mask.astype(jnp.float32)
    g = (0.5 * a_g + 0.5 * b_g) * (sample * maskf) + c_g * (ae_g * maskf)
    g_ref[...] = g.astype(jnp.bfloat16)
    col_ref[...] = jnp.sum(g, axis=1, keepdims=True)


def _attn(cnt_sd, prep, gcoef):
    qpe, kpe = prep["pe"][1], prep["pe"][2]
    qse, kse = prep["se"][1], prep["se"][2]
    qae, kae = prep["ae"][1], prep["ae"][2]
    ones = jnp.ones((N, 1), jnp.float32)
    BA = 128
    return pl.pallas_call(
        _attn_kernel,
        grid=(N // BA,),
        in_specs=[
            pl.BlockSpec((BA, C_PE), lambda i: (i, 0)),
            pl.BlockSpec((N, C_PE), lambda i: (0, 0)),
            pl.BlockSpec((BA, C_SE), lambda i: (i, 0)),
            pl.BlockSpec((N, C_SE), lambda i: (0, 0)),
            pl.BlockSpec((BA, C_AE), lambda i: (i, 0)),
            pl.BlockSpec((N, C_AE), lambda i: (0, 0)),
            pl.BlockSpec((BA, N), lambda i: (i, 0)),
            pl.BlockSpec((N, 1), lambda i: (0, 0)),
            pl.BlockSpec(memory_space=pltpu.SMEM),
        ],
        out_specs=[
            pl.BlockSpec((BA, N), lambda i: (i, 0)),
            pl.BlockSpec((BA, 1), lambda i: (i, 0)),
        ],
        out_shape=[
            jax.ShapeDtypeStruct((N, N), jnp.bfloat16),
            jax.ShapeDtypeStruct((N, 1), jnp.float32),
        ],
    )(qpe, kpe, qse, kse, qae, kae, cnt_sd, ones, gcoef)


def _final_kernel(g_ref, colrow_ref, aeh_ref, wg_ref, outl_ref, out_ref):
    gn = g_ref[...].astype(jnp.float32) / colrow_ref[...]
    out_ref[...] = outl_ref[...] + _dot(_dot(gn, aeh_ref[...]), wg_ref[...])


def _final(g, col, aeh, w_global, out_l):
    col_row = col.reshape(1, N)
    return pl.pallas_call(
        _final_kernel,
        grid=(N // BR,),
        in_specs=[
            pl.BlockSpec((BR, N), lambda i: (i, 0)),
            pl.BlockSpec((1, N), lambda i: (0, 0)),
            pl.BlockSpec((N, C_AE), lambda i: (0, 0)),
            pl.BlockSpec((C_AE, C_AE), lambda i: (0, 0)),
            pl.BlockSpec((BR, C_AE), lambda i: (i, 0)),
        ],
        out_specs=pl.BlockSpec((BR, C_AE), lambda i: (i, 0)),
        out_shape=jax.ShapeDtypeStruct((N, C_AE), jnp.float32),
    )(g, col_row, aeh, w_global, out_l)


def _tc_forward(ae, pe, se, cnt_sd, p):
    prep = _prep(ae, pe, se, p)
    lcoef = jnp.stack([p["a_l"], p["b_l"], p["c_l"]])
    gcoef = jnp.stack([p["a_g"], p["b_g"], p["c_g"]])
    out_l = _local(cnt_sd, prep, p["W_local"], lcoef)
    g, col = _attn(cnt_sd, prep, gcoef)
    return _final(g, col, prep["ae"][0], p["W_global"], out_l)


def kernel(ae, pe, se, edge_index, K, params):
    del K
    cnt_sd = _sc_cnt(edge_index)
    return _tc_forward(ae, pe, se, cnt_sd, params)


# speedup vs baseline: 1.0934x; 1.0104x over previous
import functools

import jax
import jax.numpy as jnp
from jax import lax
from jax.experimental import pallas as pl
from jax.experimental.pallas import tpu as pltpu
from jax.experimental.pallas import tpu_sc as plsc

N = 2048
E = 65536
C_AE = 256
C_PE = 64
C_SE = 64
K_TOP = 32

BR = 256
NEG = -3.0e38


_EPT = E // 16
_CHUNK = (N * N) // 4
_SLICE = _CHUNK // 16
_NDUM = 64
_ZB = 32768


def _sc_cnt_kernel(edge_hbm, sd_hbm, src_v, dst_v, idxb, valb, zb, chunk_sh, sem):
    cid = lax.axis_index("c")
    sid = lax.axis_index("s")

    ebase = sid * _EPT
    pltpu.sync_copy(edge_hbm.at[0, pl.ds(ebase, _EPT)], src_v)
    pltpu.sync_copy(edge_hbm.at[1, pl.ds(ebase, _EPT)], dst_v)

    def _zero_zb(i, _):
        zb[pl.ds(i * 16, 16)] = jnp.zeros((16,), jnp.float32)
        return 0

    lax.fori_loop(0, _ZB // 16, _zero_zb, 0)

    lanes = lax.iota(jnp.int32, 16)

    for j in range(2):
        base_dyn = (cid * 2 + j) * _CHUNK
        for t in range(_SLICE // _ZB):
            pltpu.sync_copy(zb, chunk_sh.at[pl.ds(sid * _SLICE + t * _ZB, _ZB)])

        @pl.when(sid == 0)
        def _():
            pltpu.sync_copy(zb.at[pl.ds(0, _NDUM)], chunk_sh.at[pl.ds(_CHUNK, _NDUM)])

        plsc.subcore_barrier()

        copies = []
        for g in range(32):
            def _fill(i, _, g=g):
                s16 = src_v[pl.ds(g * 128 + i * 16, 16)]
                d16 = dst_v[pl.ds(g * 128 + i * 16, 16)]
                flat = lax.shift_left(s16, 11) + d16
                rel = flat - base_dyn
                inm = (rel >= 0) & (rel < _CHUNK)
                idx16 = jnp.where(inm, rel, _CHUNK + lanes * 4)
                val16 = jnp.where(inm, jnp.float32(1.0), jnp.float32(0.0))
                idxb[g, pl.ds(i * 16, 16)] = idx16
                valb[g, pl.ds(i * 16, 16)] = val16
                return 0

            lax.fori_loop(0, 8, _fill, 0)
            copies.append(
                pltpu.async_copy(valb.at[g], chunk_sh.at[idxb.at[g]], sem, add=True)
            )
        for cp in copies:
            cp.wait()

        plsc.subcore_barrier()

        pltpu.sync_copy(chunk_sh.at[pl.ds(sid * _SLICE, _SLICE)],
                        sd_hbm.at[pl.ds(base_dyn + sid * _SLICE, _SLICE)])

        plsc.subcore_barrier()


def _sc_cnt(edge_index):
    mesh = plsc.VectorSubcoreMesh(core_axis_name="c", subcore_axis_name="s")
    f = functools.partial(
        pl.kernel,
        mesh=mesh,
        out_type=jax.ShapeDtypeStruct((N * N,), jnp.float32),
        scratch_types=[
            pltpu.VMEM((_EPT,), jnp.int32),
            pltpu.VMEM((_EPT,), jnp.int32),
            pltpu.VMEM((32, 128), jnp.int32),
            pltpu.VMEM((32, 128), jnp.float32),
            pltpu.VMEM((_ZB,), jnp.float32),
            pltpu.VMEM_SHARED((_CHUNK + _NDUM,), jnp.float32),
            pltpu.SemaphoreType.DMA,
        ],
    )(_sc_cnt_kernel)
    return f(edge_index).reshape(N, N)


def _dot(a, b):
    return lax.dot_general(a, b, (((1,), (0,)), ((), ())),
                           preferred_element_type=jnp.float32)


def _dot_hi(a, b):
    return lax.dot_general(a, b, (((1,), (0,)), ((), ())),
                           precision=lax.Precision.HIGHEST,
                           preferred_element_type=jnp.float32)


def _prep_kernel(ae_ref, pe_ref, se_ref, *rest):
    wrefs = rest[:27]
    outs = rest[27:]
    xs = (ae_ref, pe_ref, se_ref)
    for c in range(3):
        W1, b1, W2, b2, Wq, bq, Wk, bk, vsd = wrefs[c * 9:(c + 1) * 9]
        x = xs[c][...]
        h1 = jnp.maximum(_dot(x, W1[...]) + b1[...], 0.0)
        xh = _dot(h1, W2[...]) + b2[...]
        q = _dot(xh, Wq[...]) + bq[...]
        k = _dot(xh, Wk[...]) + bk[...]
        sc = _dot(xh, vsd[...])
        o_h, o_q, o_k, o_sc = outs[c * 4:(c + 1) * 4]
        o_h[...] = xh
        o_q[...] = q
        o_k[...] = k
        o_sc[...] = sc


def _prep(ae, pe, se, p):
    chans = ("ae", "pe", "se")
    dims = {"ae": C_AE, "pe": C_PE, "se": C_SE}
    weights = []
    w_specs = []
    outs = []
    o_specs = []
    for name in chans:
        c = dims[name]
        vsd = p[f"{name}_gat_W"] @ jnp.stack(
            [p[f"{name}_gat_asrc"], p[f"{name}_gat_adst"]], axis=1)
        ws = [
            p[f"{name}_enc_W1"], p[f"{name}_enc_b1"].reshape(1, 2 * c),
            p[f"{name}_enc_W2"], p[f"{name}_enc_b2"].reshape(1, c),
            p[f"{name}_mha_Wq"], p[f"{name}_mha_bq"].reshape(1, c),
            p[f"{name}_mha_Wk"], p[f"{name}_mha_bk"].reshape(1, c),
            vsd,
        ]
        weights += ws
        w_specs += [pl.BlockSpec(w.shape, lambda i: (0, 0)) for w in ws]
        outs += [
            jax.ShapeDtypeStruct((N, c), jnp.float32),
            jax.ShapeDtypeStruct((N, c), jnp.float32),
            jax.ShapeDtypeStruct((N, c), jnp.float32),
            jax.ShapeDtypeStruct((N, 2), jnp.float32),
        ]
        o_specs += [
            pl.BlockSpec((BR, c), lambda i: (i, 0)),
            pl.BlockSpec((BR, c), lambda i: (i, 0)),
            pl.BlockSpec((BR, c), lambda i: (i, 0)),
            pl.BlockSpec((BR, 2), lambda i: (i, 0)),
        ]
    x_specs = [
        pl.BlockSpec((BR, C_AE), lambda i: (i, 0)),
        pl.BlockSpec((BR, C_PE), lambda i: (i, 0)),
        pl.BlockSpec((BR, C_SE), lambda i: (i, 0)),
    ]
    res = pl.pallas_call(
        _prep_kernel,
        grid=(N // BR,),
        in_specs=x_specs + w_specs,
        out_specs=o_specs,
        out_shape=outs,
    )(ae, pe, se, *weights)
    out = {}
    for ci, name in enumerate(chans):
        out[name] = tuple(res[ci * 4:(ci + 1) * 4])
    return out


def _dot_t(a, b):
    return lax.dot_general(
        a, b, (((0,), (0,)), ((), ())),
        precision=lax.Precision.HIGHEST,
        preferred_element_type=jnp.float32)


def _local_kernel(cnt_ref, ssrc_pe, ssrc_se, ssrc_ae, sdrow_pe, sdrow_se,
                  sdrow_ae, aeh_ref, wl_ref, ones_ref, coef_ref, out_ref):
    cntb = cnt_ref[...]
    pos = cntb > 0.0
    aw = None
    coefs = (coef_ref[2], coef_ref[0], coef_ref[1])
    for ci, (ssrc, sdrow) in enumerate(((ssrc_ae, sdrow_ae), (ssrc_pe, sdrow_pe),
                                        (ssrc_se, sdrow_se))):
        e = ssrc[...] + sdrow[...]
        e = jnp.where(e > 0.0, e, 0.2 * e)
        m = jnp.max(jnp.where(pos, e, NEG), axis=0, keepdims=True)
        ex = jnp.where(pos, jnp.exp(e - m), 0.0)
        s = jnp.sum(cntb * ex, axis=0, keepdims=True)
        alpha = ex / (s + 1e-16)
        aw = coefs[ci] * alpha if aw is None else aw + coefs[ci] * alpha
    w = cntb * aw
    num = _dot_t(w, aeh_ref[...])
    deg = _dot_t(cntb, ones_ref[...])
    out_ref[...] = _dot(num / jnp.maximum(deg, 1.0), wl_ref[...])


def _local(cnt_sd, prep, w_local, lcoef):
    ssrcs = []
    sdrows = []
    for name in ("pe", "se", "ae"):
        sc = prep[name][3]
        ssrcs.append(sc[:, 0:1])
        sdrows.append(sc[:, 1].reshape(1, N))
    ones = jnp.ones((N, 1), jnp.float32)
    return pl.pallas_call(
        _local_kernel,
        grid=(N // BR,),
        in_specs=[
            pl.BlockSpec((N, BR), lambda j: (0, j)),
            pl.BlockSpec((N, 1), lambda j: (0, 0)),
            pl.BlockSpec((N, 1), lambda j: (0, 0)),
            pl.BlockSpec((N, 1), lambda j: (0, 0)),
            pl.BlockSpec((1, BR), lambda j: (0, j)),
            pl.BlockSpec((1, BR), lambda j: (0, j)),
            pl.BlockSpec((1, BR), lambda j: (0, j)),
            pl.BlockSpec((N, C_AE), lambda j: (0, 0)),
            pl.BlockSpec((C_AE, C_AE), lambda j: (0, 0)),
            pl.BlockSpec((N, 1), lambda j: (0, 0)),
            pl.BlockSpec(memory_space=pltpu.SMEM),
        ],
        out_specs=pl.BlockSpec((BR, C_AE), lambda j: (j, 0)),
        out_shape=jax.ShapeDtypeStruct((N, C_AE), jnp.float32),
    )(cnt_sd, ssrcs[0], ssrcs[1], ssrcs[2], sdrows[0], sdrows[1], sdrows[2],
      prep["ae"][0], w_local, ones, lcoef)


def _row_softmax(q, k, scale):
    s = lax.dot_general(q, k, (((1,), (1,)), ((), ())),
                        preferred_element_type=jnp.float32) * scale
    m = jnp.max(s, axis=1, keepdims=True)
    p = jnp.exp(s - m)
    return p / jnp.sum(p, axis=1, keepdims=True)


def _attn_kernel(qpe, kpe, qse, kse, qae, kae, cnt_ref, ones_ref, coef_ref,
                 g_ref, col_ref):
    br = qpe.shape[0]
    a_g = coef_ref[0]
    b_g = coef_ref[1]
    c_g = coef_ref[2]
    ones = ones_ref[...]
    pe_g = _row_softmax(qpe[...], kpe[...], 1.0 / 8.0)
    se_g = _row_softmax(qse[...], kse[...], 1.0 / 8.0)
    ae_g = _row_softmax(qae[...], kae[...], 1.0 / 16.0)
    sample = a_g * pe_g + b_g * se_g
    sample = jnp.where(cnt_ref[...] > 0.0, 0.0, sample)

    del ones

    def _count(m):
        return jnp.sum(m.astype(jnp.float32), axis=1, keepdims=True)

    v = lax.bitcast_convert_type(sample, jnp.int32)
    th = jnp.zeros((br, 1), jnp.int32)
    for b in range(29, -1, -1):
        cand = th | (1 << b)
        cge = _count(v >= cand)
        th = jnp.where(cge >= float(K_TOP), cand, th)
    gt = v > th
    eq = v == th
    ngt = _count(gt)
    need = float(K_TOP) - ngt
    cols = lax.broadcasted_iota(jnp.int32, (br, N), 1)
    jt = jnp.zeros((br, 1), jnp.int32)
    for b in range(10, -1, -1):
        cand = jt | (1 << b)
        c = _count(eq & (cols <= cand))
        jt = jnp.where(c <= need, cand, jt)
    mask = gt | (eq & (cols <= jt) & (need > 0.0))
    maskf = mask.astype(jnp.float32)
    g = (0.5 * a_g + 0.5 * b_g) * (sample * maskf) + c_g * (ae_g * maskf)
    g_ref[...] = g.astype(jnp.bfloat16)
    col_ref[...] = jnp.sum(g, axis=1, keepdims=True)


def _attn(cnt_sd, prep, gcoef):
    qpe, kpe = prep["pe"][1], prep["pe"][2]
    qse, kse = prep["se"][1], prep["se"][2]
    qae, kae = prep["ae"][1], prep["ae"][2]
    ones = jnp.ones((N, 1), jnp.float32)
    BA = 256
    return pl.pallas_call(
        _attn_kernel,
        grid=(N // BA,),
        in_specs=[
            pl.BlockSpec((BA, C_PE), lambda i: (i, 0)),
            pl.BlockSpec((N, C_PE), lambda i: (0, 0)),
            pl.BlockSpec((BA, C_SE), lambda i: (i, 0)),
            pl.BlockSpec((N, C_SE), lambda i: (0, 0)),
            pl.BlockSpec((BA, C_AE), lambda i: (i, 0)),
            pl.BlockSpec((N, C_AE), lambda i: (0, 0)),
            pl.BlockSpec((BA, N), lambda i: (i, 0)),
            pl.BlockSpec((N, 1), lambda i: (0, 0)),
            pl.BlockSpec(memory_space=pltpu.SMEM),
        ],
        out_specs=[
            pl.BlockSpec((BA, N), lambda i: (i, 0)),
            pl.BlockSpec((BA, 1), lambda i: (i, 0)),
        ],
        out_shape=[
            jax.ShapeDtypeStruct((N, N), jnp.bfloat16),
            jax.ShapeDtypeStruct((N, 1), jnp.float32),
        ],
    )(qpe, kpe, qse, kse, qae, kae, cnt_sd, ones, gcoef)


def _final_kernel(g_ref, colrow_ref, aeh_ref, wg_ref, outl_ref, out_ref):
    gn = g_ref[...].astype(jnp.float32) / colrow_ref[...]
    out_ref[...] = outl_ref[...] + _dot(_dot(gn, aeh_ref[...]), wg_ref[...])


def _final(g, col, aeh, w_global, out_l):
    col_row = col.reshape(1, N)
    return pl.pallas_call(
        _final_kernel,
        grid=(N // BR,),
        in_specs=[
            pl.BlockSpec((BR, N), lambda i: (i, 0)),
            pl.BlockSpec((1, N), lambda i: (0, 0)),
            pl.BlockSpec((N, C_AE), lambda i: (0, 0)),
            pl.BlockSpec((C_AE, C_AE), lambda i: (0, 0)),
            pl.BlockSpec((BR, C_AE), lambda i: (i, 0)),
        ],
        out_specs=pl.BlockSpec((BR, C_AE), lambda i: (i, 0)),
        out_shape=jax.ShapeDtypeStruct((N, C_AE), jnp.float32),
    )(g, col_row, aeh, w_global, out_l)


def _tc_forward(ae, pe, se, cnt_sd, p):
    prep = _prep(ae, pe, se, p)
    lcoef = jnp.stack([p["a_l"], p["b_l"], p["c_l"]])
    gcoef = jnp.stack([p["a_g"], p["b_g"], p["c_g"]])
    out_l = _local(cnt_sd, prep, p["W_local"], lcoef)
    g, col = _attn(cnt_sd, prep, gcoef)
    return _final(g, col, prep["ae"][0], p["W_global"], out_l)


def kernel(ae, pe, se, edge_index, K, params):
    del K
    cnt_sd = _sc_cnt(edge_index)
    return _tc_forward(ae, pe, se, cnt_sd, params)
